# Initial kernel scaffold; baseline (speedup 1.0000x reference)
import jax
import jax.numpy as jnp
from jax.experimental import pallas as pl


def kernel(q, edges, senders, receivers, dt, w_self, w_msg, w_edge, w_aux, w_coarse, w_gate, b):
    raise NotImplementedError("write your pallas kernel here")


# speedup vs baseline: 845.3474x; 845.3474x over previous
import functools

import jax
import jax.numpy as jnp
from jax import lax
from jax.experimental import pallas as pl
from jax.experimental.pallas import tpu as pltpu
from jax.experimental.pallas import tpu_sc as plsc

_N = 100000
_NPAD = 100096
_E = 6400000
_D = 8
_NC, _NS = 2, 16
_NW = _NC * _NS
_EW = _E // _NW
_W = 1600
_NWIN = _EW // _W
_CH = 64
_NCH = _W // _CH
_RT = _N // _NS


def _sc_body(x_hbm, snd_hbm, rcv_hbm, es_hbm, z1_hbm, z2_hbm,
             out_hbm, out2_hbm,
             idx_s, idx_r, es_v, rows, acc, acc2, gsem, ssem, s2sem):
    cid = lax.axis_index("c")
    sid = lax.axis_index("s")
    wid = cid * _NS + sid

    pltpu.sync_copy(z1_hbm.at[pl.ds(sid * _RT, _RT)],
                    acc.at[pl.ds(sid * _RT, _RT)])
    @pl.when(sid == 0)
    def _():
        pltpu.sync_copy(z2_hbm, acc2)
    plsc.subcore_barrier()

    def _window(w, _):
        base = wid * _EW + w * _W
        row0 = base // _CH
        pltpu.sync_copy(snd_hbm.at[pl.ds(row0, _NCH)], idx_s)
        pltpu.sync_copy(rcv_hbm.at[pl.ds(row0, _NCH)], idx_r)
        pltpu.sync_copy(es_hbm.at[pl.ds(base, _W)], es_v)
        gd = [pltpu.async_copy(x_hbm.at[idx_s.at[c]],
                               rows.at[pl.ds(c * _CH, _CH)], gsem)
              for c in range(_NCH)]
        for d in gd:
            d.wait()
        sd = [pltpu.async_copy(rows.at[pl.ds(c * _CH, _CH)],
                               acc.at[idx_r.at[c]], ssem, add=True)
              for c in range(_NCH)]
        s2 = [pltpu.async_copy(es_v.at[pl.ds(c * _CH, _CH)],
                               acc2.at[idx_r.at[c]], s2sem, add=True)
              for c in range(_NCH)]
        for d in sd:
            d.wait()
        for d in s2:
            d.wait()
        return _
    lax.fori_loop(0, _NWIN, _window, 0)

    plsc.subcore_barrier()
    pltpu.sync_copy(acc.at[pl.ds(sid * _RT, _RT)],
                    out_hbm.at[cid, pl.ds(sid * _RT, _RT)])
    @pl.when(sid == 0)
    def _():
        pltpu.sync_copy(acc2, out2_hbm.at[cid, pl.ds(0, _N)])


_sc_call = functools.partial(
    pl.kernel,
    out_type=(jax.ShapeDtypeStruct((_NC, _NPAD, _D), jnp.float32),
              jax.ShapeDtypeStruct((_NC, _NPAD), jnp.float32)),
    mesh=plsc.VectorSubcoreMesh(core_axis_name="c", subcore_axis_name="s"),
    scratch_types=[
        pltpu.VMEM((_NCH, _CH), jnp.int32),
        pltpu.VMEM((_NCH, _CH), jnp.int32),
        pltpu.VMEM((_W,), jnp.float32),
        pltpu.VMEM((_W, _D), jnp.float32),
        pltpu.VMEM_SHARED((_N, _D), jnp.float32),
        pltpu.VMEM_SHARED((_N,), jnp.float32),
        pltpu.SemaphoreType.DMA,
        pltpu.SemaphoreType.DMA,
        pltpu.SemaphoreType.DMA,
    ],
    compiler_params=pltpu.CompilerParams(use_tc_tiling_on_sc=False,
                                         needs_layout_passes=False),
)(_sc_body)


_NB = 4352


def _combine_body(q_ref, acc_ref, se_ref, wm_ref, g_ref, a_ref, c_ref, o_ref):
    s = acc_ref[0] + acc_ref[1]
    m = lax.dot_general(wm_ref[...], s,
                        dimension_numbers=(((1,), (1,)), ((), ())),
                        preferred_element_type=jnp.float32)
    se = se_ref[0:1] + se_ref[1:2]
    o_ref[...] = a_ref[...] * q_ref[...] + m + g_ref[...] * se + c_ref[...]


def _combine(qp, acc, se, wm, g, a, c):
    grid = (_NPAD // _NB,)
    return pl.pallas_call(
        _combine_body,
        grid=grid,
        in_specs=[
            pl.BlockSpec((8, _NB), lambda i: (0, i)),
            pl.BlockSpec((_NC, _NB, _D), lambda i: (0, i, 0)),
            pl.BlockSpec((_NC, _NB), lambda i: (0, i)),
            pl.BlockSpec((8, _D), lambda i: (0, 0)),
            pl.BlockSpec((8, 1), lambda i: (0, 0)),
            pl.BlockSpec((8, 1), lambda i: (0, 0)),
            pl.BlockSpec((8, 1), lambda i: (0, 0)),
        ],
        out_specs=pl.BlockSpec((8, _NB), lambda i: (0, i)),
        out_shape=jax.ShapeDtypeStruct((8, _NPAD), jnp.float32),
    )(qp, acc, se, wm, g, a, c)


def kernel(q, edges, senders, receivers, dt, w_self, w_msg, w_edge,
           w_aux, w_coarse, w_gate, b):
    es = edges[:, 0]
    xt = q.T
    snd2 = senders.astype(jnp.int32).reshape(-1, _CH)
    rcv2 = receivers.astype(jnp.int32).reshape(-1, _CH)
    z1 = jnp.zeros((_N, _D), jnp.float32)
    z2 = jnp.zeros((_N,), jnp.float32)

    acc, se = _sc_call(xt, snd2, rcv2, es, z1, z2)

    wm = dt[0] * w_msg[:, None] * jnp.eye(8, dtype=jnp.float32)
    g = (dt[0] * w_msg * w_edge)[:, None]
    a = (dt[0] * w_self)[:, None]
    c = (dt[0] * b)[:, None]
    qp = jnp.pad(q, ((0, 0), (0, _NPAD - _N)))

    outp = _combine(qp, acc, se, wm, g, a, c)
    return outp[:, :_N]


# speedup vs baseline: 1041.0694x; 1.2315x over previous
import functools

import jax
import jax.numpy as jnp
from jax import lax
from jax.experimental import pallas as pl
from jax.experimental.pallas import tpu as pltpu
from jax.experimental.pallas import tpu_sc as plsc

_N = 100000
_NPAD = 100096
_E = 6400000
_D = 8
_NC, _NS = 2, 16
_NW = _NC * _NS
_EW = _E // _NW
_W = 1600
_NWIN = _EW // _W
_CH = 64
_NCH = _W // _CH
_RT = _N // _NS


def _sc_body(x_hbm, snd_hbm, rcv_hbm, es_hbm, z1_hbm, z2_hbm,
             out_hbm, out2_hbm,
             idx_s0, idx_r0, es_v0, rows0, idx_s1, idx_r1, es_v1, rows1,
             acc, acc2, gsem0, gsem1, ssem0, ssem1, lsem0, lsem1):
    cid = lax.axis_index("c")
    sid = lax.axis_index("s")
    wid = cid * _NS + sid

    bufs = ((idx_s0, idx_r0, es_v0, rows0, gsem0, ssem0, lsem0),
            (idx_s1, idx_r1, es_v1, rows1, gsem1, ssem1, lsem1))

    pltpu.sync_copy(z1_hbm.at[pl.ds(sid * _RT, _RT)],
                    acc.at[pl.ds(sid * _RT, _RT)])
    @pl.when(sid == 0)
    def _():
        pltpu.sync_copy(z2_hbm, acc2)
    plsc.subcore_barrier()

    def _fire_loads(w, p):
        idx_s, idx_r, es_v, rows, gsem, ssem, lsem = bufs[p]
        base = wid * _EW + w * _W
        row0 = base // _CH
        pltpu.async_copy(snd_hbm.at[pl.ds(row0, _NCH)], idx_s, lsem)
        pltpu.async_copy(rcv_hbm.at[pl.ds(row0, _NCH)], idx_r, lsem)
        pltpu.async_copy(es_hbm.at[pl.ds(base, _W)], es_v, lsem)

    def _fire_gathers(w, p):
        idx_s, idx_r, es_v, rows, gsem, ssem, lsem = bufs[p]
        base = wid * _EW + w * _W
        row0 = base // _CH
        pltpu.make_async_copy(snd_hbm.at[pl.ds(row0, _NCH)], idx_s, lsem).wait()
        pltpu.make_async_copy(rcv_hbm.at[pl.ds(row0, _NCH)], idx_r, lsem).wait()
        pltpu.make_async_copy(es_hbm.at[pl.ds(base, _W)], es_v, lsem).wait()
        for c in range(_NCH):
            pltpu.async_copy(x_hbm.at[idx_s.at[c]],
                             rows.at[pl.ds(c * _CH, _CH)], gsem)

    def _drain_gathers(p):
        idx_s, idx_r, es_v, rows, gsem, ssem, lsem = bufs[p]
        for c in range(_NCH):
            pltpu.make_async_copy(x_hbm.at[idx_s.at[c]],
                                  rows.at[pl.ds(c * _CH, _CH)], gsem).wait()

    def _fire_scatters(w, p):
        idx_s, idx_r, es_v, rows, gsem, ssem, lsem = bufs[p]
        for c in range(_NCH):
            pltpu.async_copy(rows.at[pl.ds(c * _CH, _CH)],
                             acc.at[idx_r.at[c]], ssem, add=True)
            pltpu.async_copy(es_v.at[pl.ds(c * _CH, _CH)],
                             acc2.at[idx_r.at[c]], ssem, add=True)

    def _drain_scatters(p):
        idx_s, idx_r, es_v, rows, gsem, ssem, lsem = bufs[p]
        for c in range(_NCH):
            pltpu.make_async_copy(rows.at[pl.ds(c * _CH, _CH)],
                                  acc.at[idx_r.at[c]], ssem).wait()
            pltpu.make_async_copy(es_v.at[pl.ds(c * _CH, _CH)],
                                  acc2.at[idx_r.at[c]], ssem).wait()

    def _phase(w_cur, cur, w_nxt, nxt, drain_nxt_scatter):
        _drain_gathers(cur)
        _fire_scatters(w_cur, cur)
        if drain_nxt_scatter:
            _drain_scatters(nxt)
        _fire_loads(w_nxt, nxt)
        _fire_gathers(w_nxt, nxt)

    _fire_loads(0, 0)
    _fire_gathers(0, 0)
    _phase(0, 0, 1, 1, False)

    def _body(k, _):
        _phase(2 * k + 1, 1, 2 * k + 2, 0, True)
        _phase(2 * k + 2, 0, 2 * k + 3, 1, True)
        return _
    lax.fori_loop(0, 61, _body, 0)

    _phase(123, 1, 124, 0, True)
    _drain_gathers(0)
    _fire_scatters(124, 0)
    _drain_scatters(1)
    _drain_scatters(0)

    plsc.subcore_barrier()
    pltpu.sync_copy(acc.at[pl.ds(sid * _RT, _RT)],
                    out_hbm.at[cid, pl.ds(sid * _RT, _RT)])
    @pl.when(sid == 0)
    def _():
        pltpu.sync_copy(acc2, out2_hbm.at[cid, pl.ds(0, _N)])


_sc_call = functools.partial(
    pl.kernel,
    out_type=(jax.ShapeDtypeStruct((_NC, _NPAD, _D), jnp.float32),
              jax.ShapeDtypeStruct((_NC, _NPAD), jnp.float32)),
    mesh=plsc.VectorSubcoreMesh(core_axis_name="c", subcore_axis_name="s"),
    scratch_types=[
        pltpu.VMEM((_NCH, _CH), jnp.int32),
        pltpu.VMEM((_NCH, _CH), jnp.int32),
        pltpu.VMEM((_W,), jnp.float32),
        pltpu.VMEM((_W, _D), jnp.float32),
        pltpu.VMEM((_NCH, _CH), jnp.int32),
        pltpu.VMEM((_NCH, _CH), jnp.int32),
        pltpu.VMEM((_W,), jnp.float32),
        pltpu.VMEM((_W, _D), jnp.float32),
        pltpu.VMEM_SHARED((_N, _D), jnp.float32),
        pltpu.VMEM_SHARED((_N,), jnp.float32),
        pltpu.SemaphoreType.DMA,
        pltpu.SemaphoreType.DMA,
        pltpu.SemaphoreType.DMA,
        pltpu.SemaphoreType.DMA,
        pltpu.SemaphoreType.DMA,
        pltpu.SemaphoreType.DMA,
    ],
    compiler_params=pltpu.CompilerParams(use_tc_tiling_on_sc=False,
                                         needs_layout_passes=False),
)(_sc_body)


_NB = 4352


def _combine_body(q_ref, acc_ref, se_ref, wm_ref, g_ref, a_ref, c_ref, o_ref):
    s = acc_ref[0] + acc_ref[1]
    m = lax.dot_general(wm_ref[...], s,
                        dimension_numbers=(((1,), (1,)), ((), ())),
                        preferred_element_type=jnp.float32)
    se = se_ref[0:1] + se_ref[1:2]
    o_ref[...] = a_ref[...] * q_ref[...] + m + g_ref[...] * se + c_ref[...]


def _combine(qp, acc, se, wm, g, a, c):
    grid = (_NPAD // _NB,)
    return pl.pallas_call(
        _combine_body,
        grid=grid,
        in_specs=[
            pl.BlockSpec((8, _NB), lambda i: (0, i)),
            pl.BlockSpec((_NC, _NB, _D), lambda i: (0, i, 0)),
            pl.BlockSpec((_NC, _NB), lambda i: (0, i)),
            pl.BlockSpec((8, _D), lambda i: (0, 0)),
            pl.BlockSpec((8, 1), lambda i: (0, 0)),
            pl.BlockSpec((8, 1), lambda i: (0, 0)),
            pl.BlockSpec((8, 1), lambda i: (0, 0)),
        ],
        out_specs=pl.BlockSpec((8, _NB), lambda i: (0, i)),
        out_shape=jax.ShapeDtypeStruct((8, _NPAD), jnp.float32),
    )(qp, acc, se, wm, g, a, c)


def kernel(q, edges, senders, receivers, dt, w_self, w_msg, w_edge,
           w_aux, w_coarse, w_gate, b):
    es = edges[:, 0]
    xt = q.T
    snd2 = senders.astype(jnp.int32).reshape(-1, _CH)
    rcv2 = receivers.astype(jnp.int32).reshape(-1, _CH)
    z1 = jnp.zeros((_N, _D), jnp.float32)
    z2 = jnp.zeros((_N,), jnp.float32)

    acc, se = _sc_call(xt, snd2, rcv2, es, z1, z2)

    wm = dt[0] * w_msg[:, None] * jnp.eye(8, dtype=jnp.float32)
    g = (dt[0] * w_msg * w_edge)[:, None]
    a = (dt[0] * w_self)[:, None]
    c = (dt[0] * b)[:, None]
    qp = jnp.pad(q, ((0, 0), (0, _NPAD - _N)))

    outp = _combine(qp, acc, se, wm, g, a, c)
    return outp[:, :_N]
